# baseline (speedup 1.0000x reference)
import math

import jax
import jax.numpy as jnp
from jax import lax
from jax.experimental import pallas as pl
from jax.experimental.pallas import tpu as pltpu
from jax.experimental.pallas import tpu_sc as plsc

_LANES = 16


def _positional_encoding_2d(seq_len, d):
    pos = jnp.arange(seq_len, dtype=jnp.float32)[:, None]
    even_idx = jnp.arange(0, d, 2, dtype=jnp.float32)
    odd_idx = jnp.arange(1, d, 2, dtype=jnp.float32)
    even_div = jnp.power(10000.0, 2.0 * even_idx / d)
    odd_div = jnp.power(10000.0, 2.0 * odd_idx / d)
    pe = jnp.zeros((seq_len, d), dtype=jnp.float32)
    pe = pe.at[:, 0::2].set(jnp.sin(pos / even_div))
    pe = pe.at[:, 1::2].set(jnp.cos(pos / odd_div))
    return pe


def kernel(x, table):
    b, s = x.shape
    v, d = table.shape
    scale = math.sqrt(d)
    r = b * s

    info = plsc.get_sparse_core_info()
    nc, ns = info.num_cores, info.num_subcores
    nw = nc * ns

    cr = 128
    assert r % (nw * cr) == 0 and d % _LANES == 0
    rpw = r // nw
    nch = rpw // cr
    assert nch % 4 == 0
    groups = d // _LANES

    pe2 = jnp.concatenate([_positional_encoding_2d(s, d)] * 2, axis=0).reshape(-1)
    xr = x.astype(jnp.int32).reshape(nw, nch, cr)

    mesh = plsc.VectorSubcoreMesh(core_axis_name="c", subcore_axis_name="s")

    def body(x_hbm, pe_hbm, table_hbm, out_hbm,
             idx_v, pe_v, buf0, buf1, buf2, buf3,
             gsem0, gsem1, gsem2, gsem3, wsem0, wsem1, wsem2, wsem3):
        wid = lax.axis_index("s") * nc + lax.axis_index("c")
        pltpu.sync_copy(x_hbm.at[wid], idx_v)
        pltpu.sync_copy(pe_hbm, pe_v)
        row_base = wid * rpw

        bufs = (buf0, buf1, buf2, buf3)
        gsems = (gsem0, gsem1, gsem2, gsem3)
        wsems = (wsem0, wsem1, wsem2, wsem3)

        def gather(m, bb, gsem):
            return pltpu.make_async_copy(table_hbm.at[idx_v.at[m]], bb, gsem)

        def write(m, bb, wsem):
            return pltpu.make_async_copy(
                bb, out_hbm.at[pl.ds(row_base + m * cr, cr)], wsem)

        for p in range(2):
            gather(p, bufs[p], gsems[p]).start()

        @pl.loop(0, nch // 4)
        def _outer(t):
            for p in range(4):
                m = 4 * t + p
                gather(m, bufs[p], gsems[p]).wait()
                pe_base = lax.rem(m * cr, s) * d

                @pl.loop(0, cr, step=2)
                def _token(j0):
                    offs = [(j0 + tj) * d + fb * _LANES
                            for tj in (0, 1) for fb in range(groups)]
                    vecs = [bufs[p][(j0 + tj), pl.ds(fb * _LANES, _LANES)]
                            for tj in (0, 1) for fb in range(groups)]
                    pes = [pe_v[pl.ds(pe_base + o, _LANES)] for o in offs]
                    scaled = [vv * scale for vv in vecs]
                    added = [sc + pp for sc, pp in zip(scaled, pes)]
                    for k, (tj, fb) in enumerate(
                            [(tj, fb) for tj in (0, 1) for fb in range(groups)]):
                        bufs[p][j0 + tj, pl.ds(fb * _LANES, _LANES)] = added[k]

                write(m, bufs[p], wsems[p]).start()

                pn = (p + 2) % 4
                @pl.when(m + 2 < nch)
                def _next_gather():
                    @pl.when(m >= 2)
                    def _drain():
                        write(m - 2, bufs[pn], wsems[pn]).wait()
                    gather(m + 2, bufs[pn], gsems[pn]).start()

        for p in range(4):
            write(nch - 4 + p, bufs[p], wsems[p]).wait()

    out_flat = pl.kernel(
        body,
        out_type=jax.ShapeDtypeStruct((r, d), jnp.float32),
        mesh=mesh,
        compiler_params=pltpu.CompilerParams(use_tc_tiling_on_sc=False,
                                             needs_layout_passes=False),
        scratch_types=[
            pltpu.VMEM((nch, cr), jnp.int32),
            pltpu.VMEM((2 * s * d,), jnp.float32),
            pltpu.VMEM((cr, d), jnp.float32),
            pltpu.VMEM((cr, d), jnp.float32),
            pltpu.VMEM((cr, d), jnp.float32),
            pltpu.VMEM((cr, d), jnp.float32),
            pltpu.SemaphoreType.DMA,
            pltpu.SemaphoreType.DMA,
            pltpu.SemaphoreType.DMA,
            pltpu.SemaphoreType.DMA,
            pltpu.SemaphoreType.DMA,
            pltpu.SemaphoreType.DMA,
            pltpu.SemaphoreType.DMA,
            pltpu.SemaphoreType.DMA,
        ],
    )(xr, pe2, table)
    return out_flat.reshape(b, s, d)
